# baseline (speedup 1.0000x reference)
import functools

import jax
import jax.numpy as jnp
from jax import lax
from jax.experimental import pallas as pl
from jax.experimental.pallas import tpu as pltpu
from jax.experimental.pallas import tpu_sc as plsc

N = 100000
D = 128
NSEG = 256
NT = 16
DC = D // 2
CHUNK = 256
CPT = 25
GROUP = 32
LAST_START = N - CHUNK
NEG_INF = float("-inf")


def _segmax(feat, ids):
    mesh = plsc.VectorSubcoreMesh(core_axis_name="c", subcore_axis_name="s")

    @functools.partial(
        pl.kernel,
        out_type=jax.ShapeDtypeStruct((NSEG, 2, DC), jnp.float32),
        mesh=mesh,
        scratch_types=[
            pltpu.VMEM((CHUNK, DC), jnp.float32),
            pltpu.VMEM((CHUNK, DC), jnp.float32),
            pltpu.VMEM((CHUNK,), jnp.int32),
            pltpu.VMEM((CHUNK,), jnp.int32),
            pltpu.VMEM((NSEG * DC,), jnp.float32),
            pltpu.VMEM((NT, NSEG * DC // NT), jnp.float32),
            pltpu.VMEM((NSEG // NT, DC), jnp.float32),
            pltpu.VMEM_SHARED((NT, NSEG * DC), jnp.float32),
            pltpu.SemaphoreType.DMA,
            pltpu.SemaphoreType.DMA,
            pltpu.SemaphoreType.DMA,
            pltpu.SemaphoreType.DMA,
        ],
    )
    def k(feat_hbm, ids_hbm, out_hbm, buf0, buf1, idv0, idv1, table,
          cbuf, obuf, spm, sf0, sf1, si0, si1):
        core = lax.axis_index("c")
        t = lax.axis_index("s")
        neg = jnp.full((16,), NEG_INF, jnp.float32)
        JW = DC // 16

        def row0_of(c):
            return jnp.minimum((t + c * NT) * CHUNK, LAST_START)

        def fetch(c, buf, idv, semf, semi):
            r0 = row0_of(c)
            return (
                pltpu.make_async_copy(
                    feat_hbm.at[pl.ds(r0, CHUNK), core, :], buf, semf),
                pltpu.make_async_copy(
                    ids_hbm.at[pl.ds(r0, CHUNK)], idv, semi),
            )

        def start_fetch(c, buf, idv, semf, semi):
            for cp in fetch(c, buf, idv, semf, semi):
                cp.start()

        def wait_fetch(c, buf, idv, semf, semi):
            for cp in fetch(c, buf, idv, semf, semi):
                cp.wait()

        start_fetch(0, buf0, idv0, sf0, si0)

        def init(i, _):
            table[pl.ds(i * 16, 16)] = neg
            return 0

        lax.fori_loop(0, NSEG * DC // 16, init, 0)

        def flush(prev, accs):
            for j in range(JW):
                tt = table[pl.ds(prev * DC + j * 16, 16)]
                table[pl.ds(prev * DC + j * 16, 16)] = jnp.maximum(tt, accs[j])

        def process(buf, idv, carry):
            def group(g, carry):
                prev = carry[0]
                accs = carry[1:]
                r0 = g * GROUP
                ids_a = idv[pl.ds(r0, 16)]
                ids_b = idv[pl.ds(r0 + 16, 16)]
                first = ids_a[0]
                last = ids_b[15]
                uniform = jnp.logical_and(first == prev, last == prev)

                acc_fast = []
                for j in range(JW):
                    a = accs[j]
                    for l in range(GROUP):
                        a = jnp.maximum(a, buf[r0 + l, pl.ds(j * 16, 16)])
                    acc_fast.append(a)

                @pl.when(jnp.logical_not(uniform))
                def _():
                    @pl.when(prev >= 0)
                    def _():
                        flush(prev, accs)

                    for l in range(GROUP):
                        sid = ids_a[l] if l < 16 else ids_b[l - 16]
                        for j in range(JW):
                            tt = table[pl.ds(sid * DC + j * 16, 16)]
                            v = buf[r0 + l, pl.ds(j * 16, 16)]
                            table[pl.ds(sid * DC + j * 16, 16)] = (
                                jnp.maximum(tt, v))

                new_prev = jnp.where(uniform, prev, last)
                new_accs = [jnp.where(uniform, acc_fast[j], neg)
                            for j in range(JW)]
                return (new_prev, *new_accs)

            return lax.fori_loop(0, CHUNK // GROUP, group, carry)

        carry = (jnp.int32(-1), *([neg] * JW))

        def pair_body(g, carry):
            c0 = 2 * g
            start_fetch(c0 + 1, buf1, idv1, sf1, si1)
            wait_fetch(c0, buf0, idv0, sf0, si0)
            carry = process(buf0, idv0, carry)
            start_fetch(c0 + 2, buf0, idv0, sf0, si0)
            wait_fetch(c0 + 1, buf1, idv1, sf1, si1)
            carry = process(buf1, idv1, carry)
            return carry

        carry = lax.fori_loop(0, (CPT - 1) // 2, pair_body, carry)
        wait_fetch(CPT - 1, buf0, idv0, sf0, si0)
        carry = process(buf0, idv0, carry)

        prev = carry[0]

        @pl.when(prev >= 0)
        def _():
            flush(prev, carry[1:])

        pltpu.sync_copy(table, spm.at[t])
        plsc.subcore_barrier()

        SLICE = NSEG * DC // NT
        pltpu.sync_copy(spm.at[:, pl.ds(t * SLICE, SLICE)], cbuf)

        def red(s, _):
            for j in range(JW):
                m = neg
                for i in range(NT):
                    m = jnp.maximum(m, cbuf[i, pl.ds(s * DC + j * 16, 16)])
                obuf[s, pl.ds(j * 16, 16)] = m
            return 0

        lax.fori_loop(0, NSEG // NT, red, 0)
        pltpu.sync_copy(
            obuf, out_hbm.at[pl.ds(t * (NSEG // NT), NSEG // NT), core, :])

    return k(feat, ids)


def kernel(feat, segment_ids):
    feat3 = feat.reshape(N, 2, DC)
    out = _segmax(feat3, segment_ids.astype(jnp.int32))
    return out.reshape(NSEG, D)


# speedup vs baseline: 2.4236x; 2.4236x over previous
import functools

import jax
import jax.numpy as jnp
from jax import lax
from jax.experimental import pallas as pl
from jax.experimental.pallas import tpu as pltpu
from jax.experimental.pallas import tpu_sc as plsc

N = 100000
D = 128
NSEG = 256
NW = 32
CHUNK = 256
CPW = 13
LAST_START = N - CHUNK
NEG_INF = float("-inf")


def _phase1(feat1d, ids):
    mesh = plsc.VectorSubcoreMesh(core_axis_name="c", subcore_axis_name="s")

    @functools.partial(
        pl.kernel,
        out_type=jax.ShapeDtypeStruct((NW, NSEG * D), jnp.float32),
        mesh=mesh,
        scratch_types=[
            pltpu.VMEM((CHUNK * D,), jnp.float32),
            pltpu.VMEM((CHUNK * D,), jnp.float32),
            pltpu.VMEM((CHUNK,), jnp.int32),
            pltpu.VMEM((CHUNK,), jnp.int32),
            pltpu.VMEM((NSEG * D,), jnp.float32),
            pltpu.SemaphoreType.DMA,
            pltpu.SemaphoreType.DMA,
            pltpu.SemaphoreType.DMA,
            pltpu.SemaphoreType.DMA,
        ],
    )
    def k(feat_hbm, ids_hbm, part_hbm, buf0, buf1, idv0, idv1, table,
          sf0, sf1, si0, si1):
        wid = lax.axis_index("s") * 2 + lax.axis_index("c")
        neg = jnp.full((16,), NEG_INF, jnp.float32)

        def row0_of(c):
            return jnp.minimum((wid + c * NW) * CHUNK, LAST_START)

        def fetch(c, buf, idv, semf, semi):
            r0 = row0_of(c)
            return (
                pltpu.make_async_copy(
                    feat_hbm.at[pl.ds(r0 * D, CHUNK * D)], buf, semf),
                pltpu.make_async_copy(
                    ids_hbm.at[pl.ds(r0, CHUNK)], idv, semi),
            )

        def start_fetch(c, buf, idv, semf, semi):
            for cp in fetch(c, buf, idv, semf, semi):
                cp.start()

        def wait_fetch(c, buf, idv, semf, semi):
            for cp in fetch(c, buf, idv, semf, semi):
                cp.wait()

        start_fetch(0, buf0, idv0, sf0, si0)

        def init(i, _):
            table[pl.ds(i * 16, 16)] = neg
            return 0

        lax.fori_loop(0, NSEG * D // 16, init, 0)

        def flush(prev, accs):
            for j in range(D // 16):
                t = table[pl.ds(prev * D + j * 16, 16)]
                table[pl.ds(prev * D + j * 16, 16)] = jnp.maximum(t, accs[j])

        def process(buf, idv, carry):
            def group(g, carry):
                prev = carry[0]
                accs = carry[1:]
                r0 = g * 32
                ids_a = idv[pl.ds(r0, 16)]
                ids_b = idv[pl.ds(r0 + 16, 16)]
                first = ids_a[0]
                last = ids_b[15]
                uniform = jnp.logical_and(first == prev, last == prev)

                acc_fast = []
                for j in range(D // 16):
                    a = accs[j]
                    for l in range(32):
                        a = jnp.maximum(
                            a, buf[pl.ds((r0 + l) * D + j * 16, 16)])
                    acc_fast.append(a)

                @pl.when(jnp.logical_not(uniform))
                def _():
                    @pl.when(prev >= 0)
                    def _():
                        flush(prev, accs)

                    for l in range(32):
                        sid = ids_a[l] if l < 16 else ids_b[l - 16]
                        for j in range(D // 16):
                            t = table[pl.ds(sid * D + j * 16, 16)]
                            v = buf[pl.ds((r0 + l) * D + j * 16, 16)]
                            table[pl.ds(sid * D + j * 16, 16)] = jnp.maximum(t, v)

                new_prev = jnp.where(uniform, prev, last)
                new_accs = [jnp.where(uniform, acc_fast[j], neg)
                            for j in range(D // 16)]
                return (new_prev, *new_accs)

            return lax.fori_loop(0, CHUNK // 32, group, carry)

        carry = (jnp.int32(-1), *([neg] * (D // 16)))

        def pair_body(g, carry):
            c0 = 2 * g
            start_fetch(c0 + 1, buf1, idv1, sf1, si1)
            wait_fetch(c0, buf0, idv0, sf0, si0)
            carry = process(buf0, idv0, carry)
            start_fetch(c0 + 2, buf0, idv0, sf0, si0)
            wait_fetch(c0 + 1, buf1, idv1, sf1, si1)
            carry = process(buf1, idv1, carry)
            return carry

        carry = lax.fori_loop(0, (CPW - 1) // 2, pair_body, carry)
        wait_fetch(CPW - 1, buf0, idv0, sf0, si0)
        carry = process(buf0, idv0, carry)

        prev = carry[0]

        @pl.when(prev >= 0)
        def _():
            flush(prev, carry[1:])

        pltpu.sync_copy(table, part_hbm.at[wid])

    return k(feat1d, ids)


def _phase2(part):
    mesh = plsc.VectorSubcoreMesh(core_axis_name="c", subcore_axis_name="s")
    COLS = NSEG * D // NW

    @functools.partial(
        pl.kernel,
        out_type=jax.ShapeDtypeStruct((NSEG * D,), jnp.float32),
        mesh=mesh,
        scratch_types=[
            pltpu.VMEM((NW, COLS), jnp.float32),
            pltpu.VMEM((COLS,), jnp.float32),
        ],
    )
    def k(part_hbm, out_hbm, pbuf, obuf):
        wid = lax.axis_index("s") * 2 + lax.axis_index("c")
        col0 = wid * COLS
        pltpu.sync_copy(part_hbm.at[:, pl.ds(col0, COLS)], pbuf)

        def col_body(t, _):
            m = jnp.full((16,), NEG_INF, jnp.float32)
            for i in range(NW):
                m = jnp.maximum(m, pbuf[i, pl.ds(t * 16, 16)])
            obuf[pl.ds(t * 16, 16)] = m
            return 0

        lax.fori_loop(0, COLS // 16, col_body, 0)
        pltpu.sync_copy(obuf, out_hbm.at[pl.ds(col0, COLS)])

    return k(part)


def kernel(feat, segment_ids):
    feat1d = feat.reshape(-1)
    ids = segment_ids.astype(jnp.int32)
    part = _phase1(feat1d, ids)
    out = _phase2(part)
    return out.reshape(NSEG, D)


# speedup vs baseline: 3.0745x; 1.2686x over previous
import functools

import jax
import jax.numpy as jnp
from jax import lax
from jax.experimental import pallas as pl
from jax.experimental.pallas import tpu as pltpu
from jax.experimental.pallas import tpu_sc as plsc

N = 100000
D = 128
NSEG = 256
NW = 32
CHUNK = 256
CPW = 13
LAST_START = N - CHUNK
NEG_INF = float("-inf")


def _phase1(feat1d, ids):
    mesh = plsc.VectorSubcoreMesh(core_axis_name="c", subcore_axis_name="s")

    @functools.partial(
        pl.kernel,
        out_type=jax.ShapeDtypeStruct((NW, NSEG * D), jnp.float32),
        mesh=mesh,
        scratch_types=[
            pltpu.VMEM((CHUNK, D), jnp.float32),
            pltpu.VMEM((CHUNK, D), jnp.float32),
            pltpu.VMEM((CHUNK,), jnp.int32),
            pltpu.VMEM((CHUNK,), jnp.int32),
            pltpu.VMEM((NSEG * D,), jnp.float32),
            pltpu.SemaphoreType.DMA,
            pltpu.SemaphoreType.DMA,
            pltpu.SemaphoreType.DMA,
            pltpu.SemaphoreType.DMA,
        ],
    )
    def k(feat_hbm, ids_hbm, part_hbm, buf0, buf1, idv0, idv1, table,
          sf0, sf1, si0, si1):
        wid = lax.axis_index("s") * 2 + lax.axis_index("c")
        neg = jnp.full((16,), NEG_INF, jnp.float32)

        def row0_of(c):
            return jnp.minimum((wid + c * NW) * CHUNK, LAST_START)

        def fetch(c, buf, idv, semf, semi):
            r0 = row0_of(c)
            return (
                pltpu.make_async_copy(
                    feat_hbm.at[pl.ds(r0, CHUNK), :], buf, semf),
                pltpu.make_async_copy(
                    ids_hbm.at[pl.ds(r0, CHUNK)], idv, semi),
            )

        def start_fetch(c, buf, idv, semf, semi):
            for cp in fetch(c, buf, idv, semf, semi):
                cp.start()

        def wait_fetch(c, buf, idv, semf, semi):
            for cp in fetch(c, buf, idv, semf, semi):
                cp.wait()

        start_fetch(0, buf0, idv0, sf0, si0)

        def init(i, _):
            table[pl.ds(i * 16, 16)] = neg
            return 0

        lax.fori_loop(0, NSEG * D // 16, init, 0)

        def flush(prev, accs):
            for j in range(D // 16):
                t = table[pl.ds(prev * D + j * 16, 16)]
                table[pl.ds(prev * D + j * 16, 16)] = jnp.maximum(t, accs[j])

        def process(buf, idv, carry):
            def group(rb, carry):
                prev = carry[0]
                accs = carry[1:]
                ids16 = idv[pl.ds(rb * 16, 16)]
                first = ids16[0]
                last = ids16[15]
                uniform = jnp.logical_and(first == prev, last == prev)

                acc_fast = []
                for j in range(D // 16):
                    a = accs[j]
                    for l in range(16):
                        a = jnp.maximum(
                            a, buf[rb * 16 + l, pl.ds(j * 16, 16)])
                    acc_fast.append(a)

                @pl.when(jnp.logical_not(uniform))
                def _():
                    @pl.when(prev >= 0)
                    def _():
                        flush(prev, accs)

                    for l in range(16):
                        sid = ids16[l]
                        for j in range(D // 16):
                            t = table[pl.ds(sid * D + j * 16, 16)]
                            v = buf[rb * 16 + l, pl.ds(j * 16, 16)]
                            table[pl.ds(sid * D + j * 16, 16)] = jnp.maximum(t, v)

                new_prev = jnp.where(uniform, prev, last)
                new_accs = [jnp.where(uniform, acc_fast[j], neg)
                            for j in range(D // 16)]
                return (new_prev, *new_accs)

            return lax.fori_loop(0, CHUNK // 16, group, carry)

        carry = (jnp.int32(-1), *([neg] * (D // 16)))

        def pair_body(g, carry):
            c0 = 2 * g
            start_fetch(c0 + 1, buf1, idv1, sf1, si1)
            wait_fetch(c0, buf0, idv0, sf0, si0)
            carry = process(buf0, idv0, carry)
            start_fetch(c0 + 2, buf0, idv0, sf0, si0)
            wait_fetch(c0 + 1, buf1, idv1, sf1, si1)
            carry = process(buf1, idv1, carry)
            return carry

        carry = lax.fori_loop(0, (CPW - 1) // 2, pair_body, carry)
        wait_fetch(CPW - 1, buf0, idv0, sf0, si0)
        carry = process(buf0, idv0, carry)

        prev = carry[0]

        @pl.when(prev >= 0)
        def _():
            flush(prev, carry[1:])

        pltpu.sync_copy(table, part_hbm.at[wid])

    return k(feat1d, ids)


def _phase2(part):
    mesh = plsc.VectorSubcoreMesh(core_axis_name="c", subcore_axis_name="s")
    COLS = NSEG * D // NW

    SPW = NSEG // NW

    @functools.partial(
        pl.kernel,
        out_type=jax.ShapeDtypeStruct((NSEG, D), jnp.float32),
        mesh=mesh,
        scratch_types=[
            pltpu.VMEM((NW, COLS), jnp.float32),
            pltpu.VMEM((SPW, D), jnp.float32),
        ],
    )
    def k(part_hbm, out_hbm, pbuf, obuf):
        wid = lax.axis_index("s") * 2 + lax.axis_index("c")
        col0 = wid * COLS
        pltpu.sync_copy(part_hbm.at[:, pl.ds(col0, COLS)], pbuf)

        def seg_body(s, _):
            for j in range(D // 16):
                m = jnp.full((16,), NEG_INF, jnp.float32)
                for i in range(NW):
                    m = jnp.maximum(m, pbuf[i, pl.ds(s * D + j * 16, 16)])
                obuf[s, pl.ds(j * 16, 16)] = m
            return 0

        lax.fori_loop(0, SPW, seg_body, 0)
        pltpu.sync_copy(obuf, out_hbm.at[pl.ds(wid * SPW, SPW), :])

    return k(part)


def kernel(feat, segment_ids):
    return _phase2(_phase1(feat, segment_ids.astype(jnp.int32)))


# speedup vs baseline: 3.1264x; 1.0169x over previous
import functools

import jax
import jax.numpy as jnp
from jax import lax
from jax.experimental import pallas as pl
from jax.experimental.pallas import tpu as pltpu
from jax.experimental.pallas import tpu_sc as plsc

N = 100000
D = 128
NSEG = 256
NW = 32
CHUNK = 256
CPW = 13
LAST_START = N - CHUNK
NEG_INF = float("-inf")


def _phase1(feat1d, ids):
    mesh = plsc.VectorSubcoreMesh(core_axis_name="c", subcore_axis_name="s")

    @functools.partial(
        pl.kernel,
        out_type=jax.ShapeDtypeStruct((NW, NSEG * D), jnp.float32),
        mesh=mesh,
        scratch_types=[
            pltpu.VMEM((CHUNK * D,), jnp.float32),
            pltpu.VMEM((CHUNK * D,), jnp.float32),
            pltpu.VMEM((CHUNK,), jnp.int32),
            pltpu.VMEM((CHUNK,), jnp.int32),
            pltpu.VMEM((NSEG * D,), jnp.float32),
            pltpu.SemaphoreType.DMA,
            pltpu.SemaphoreType.DMA,
            pltpu.SemaphoreType.DMA,
            pltpu.SemaphoreType.DMA,
        ],
    )
    def k(feat_hbm, ids_hbm, part_hbm, buf0, buf1, idv0, idv1, table,
          sf0, sf1, si0, si1):
        wid = lax.axis_index("s") * 2 + lax.axis_index("c")
        neg = jnp.full((16,), NEG_INF, jnp.float32)

        def row0_of(c):
            return jnp.minimum((wid + c * NW) * CHUNK, LAST_START)

        def fetch(c, buf, idv, semf, semi):
            r0 = row0_of(c)
            return (
                pltpu.make_async_copy(
                    feat_hbm.at[pl.ds(r0 * D, CHUNK * D)], buf, semf),
                pltpu.make_async_copy(
                    ids_hbm.at[pl.ds(r0, CHUNK)], idv, semi),
            )

        def start_fetch(c, buf, idv, semf, semi):
            for cp in fetch(c, buf, idv, semf, semi):
                cp.start()

        def wait_fetch(c, buf, idv, semf, semi):
            for cp in fetch(c, buf, idv, semf, semi):
                cp.wait()

        start_fetch(0, buf0, idv0, sf0, si0)

        def init(i, _):
            table[pl.ds(i * 16, 16)] = neg
            return 0

        lax.fori_loop(0, NSEG * D // 16, init, 0)

        def flush(prev, accs):
            for j in range(D // 16):
                t = table[pl.ds(prev * D + j * 16, 16)]
                table[pl.ds(prev * D + j * 16, 16)] = jnp.maximum(t, accs[j])

        def process(buf, idv, carry):
            def group(rb, carry):
                prev = carry[0]
                accs = carry[1:]
                ids16 = idv[pl.ds(rb * 16, 16)]
                first = ids16[0]
                last = ids16[15]
                uniform = jnp.logical_and(first == prev, last == prev)

                acc_fast = []
                for j in range(D // 16):
                    a = accs[j]
                    for l in range(16):
                        a = jnp.maximum(
                            a, buf[pl.ds((rb * 16 + l) * D + j * 16, 16)])
                    acc_fast.append(a)

                @pl.when(jnp.logical_not(uniform))
                def _():
                    @pl.when(prev >= 0)
                    def _():
                        flush(prev, accs)

                    for l in range(16):
                        sid = ids16[l]
                        for j in range(D // 16):
                            t = table[pl.ds(sid * D + j * 16, 16)]
                            v = buf[pl.ds((rb * 16 + l) * D + j * 16, 16)]
                            table[pl.ds(sid * D + j * 16, 16)] = jnp.maximum(t, v)

                new_prev = jnp.where(uniform, prev, last)
                new_accs = [jnp.where(uniform, acc_fast[j], neg)
                            for j in range(D // 16)]
                return (new_prev, *new_accs)

            return lax.fori_loop(0, CHUNK // 16, group, carry)

        carry = (jnp.int32(-1), *([neg] * (D // 16)))

        def pair_body(g, carry):
            c0 = 2 * g
            start_fetch(c0 + 1, buf1, idv1, sf1, si1)
            wait_fetch(c0, buf0, idv0, sf0, si0)
            carry = process(buf0, idv0, carry)
            start_fetch(c0 + 2, buf0, idv0, sf0, si0)
            wait_fetch(c0 + 1, buf1, idv1, sf1, si1)
            carry = process(buf1, idv1, carry)
            return carry

        carry = lax.fori_loop(0, (CPW - 1) // 2, pair_body, carry)
        wait_fetch(CPW - 1, buf0, idv0, sf0, si0)
        carry = process(buf0, idv0, carry)

        prev = carry[0]

        @pl.when(prev >= 0)
        def _():
            flush(prev, carry[1:])

        pltpu.sync_copy(table, part_hbm.at[wid])

    return k(feat1d, ids)


def _phase2(part):
    mesh = plsc.VectorSubcoreMesh(core_axis_name="c", subcore_axis_name="s")
    COLS = NSEG * D // NW

    @functools.partial(
        pl.kernel,
        out_type=jax.ShapeDtypeStruct((NSEG * D,), jnp.float32),
        mesh=mesh,
        scratch_types=[
            pltpu.VMEM((NW, COLS), jnp.float32),
            pltpu.VMEM((COLS,), jnp.float32),
        ],
    )
    def k(part_hbm, out_hbm, pbuf, obuf):
        wid = lax.axis_index("s") * 2 + lax.axis_index("c")
        col0 = wid * COLS
        pltpu.sync_copy(part_hbm.at[:, pl.ds(col0, COLS)], pbuf)

        def col_body(t, _):
            m = jnp.full((16,), NEG_INF, jnp.float32)
            for i in range(NW):
                m = jnp.maximum(m, pbuf[i, pl.ds(t * 16, 16)])
            obuf[pl.ds(t * 16, 16)] = m
            return 0

        lax.fori_loop(0, COLS // 16, col_body, 0)
        pltpu.sync_copy(obuf, out_hbm.at[pl.ds(col0, COLS)])

    return k(part)


def kernel(feat, segment_ids):
    feat1d = feat.reshape(-1)
    ids = segment_ids.astype(jnp.int32)
    part = _phase1(feat1d, ids)
    out = _phase2(part)
    return out.reshape(NSEG, D)
